# baseline (speedup 1.0000x reference)
import functools

import jax
import jax.numpy as jnp
from jax.experimental import pallas as pl
from jax.experimental.pallas import tpu as pltpu

N = 4096
C = 768
E = 64
H = 3072
M = 128
NB = N // M + E
P = NB * M
RB = 512


def _router_body(x_ref, wr_ref, br_ref, idx_ref, gate_ref):
    logits = jnp.dot(x_ref[...], wr_ref[...],
                     preferred_element_type=jnp.float32) + br_ref[...]
    m = jnp.max(logits, axis=-1, keepdims=True)
    s = jnp.sum(jnp.exp(logits - m), axis=-1, keepdims=True)
    idx_ref[...] = jnp.argmax(logits, axis=-1).astype(jnp.int32)
    gate_ref[...] = (1.0 / s)[:, 0]


def _router(flat_x, Wr, br):
    return pl.pallas_call(
        _router_body,
        grid=(N // RB,),
        in_specs=[
            pl.BlockSpec((RB, C), lambda i: (i, 0)),
            pl.BlockSpec((C, E), lambda i: (0, 0)),
            pl.BlockSpec((E,), lambda i: (0,)),
        ],
        out_specs=[
            pl.BlockSpec((RB,), lambda i: (i,)),
            pl.BlockSpec((RB,), lambda i: (i,)),
        ],
        out_shape=[
            jax.ShapeDtypeStruct((N,), jnp.int32),
            jax.ShapeDtypeStruct((N,), jnp.float32),
        ],
    )(flat_x, Wr, br)


HS = 4
HB = H // HS


def _expert_body(be_ref, na_ref, x_ref, w1_ref, b1_ref, w2_ref, b2_ref,
                 g_ref, y_ref):
    b = pl.program_id(0)
    hb = pl.program_id(1)

    @pl.when(b < na_ref[0])
    def _():
        h = jnp.dot(x_ref[...], w1_ref[0],
                    preferred_element_type=jnp.float32) + b1_ref[0]
        h = 0.5 * h * (1.0 + jax.lax.erf(h * 0.7071067811865476))
        yp = jnp.dot(h, w2_ref[0], preferred_element_type=jnp.float32)

        @pl.when(hb == 0)
        def _():
            y_ref[...] = yp + b2_ref[0]

        @pl.when(hb != 0)
        def _():
            y_ref[...] += yp

        @pl.when(hb == HS - 1)
        def _():
            y_ref[...] *= g_ref[...]


def _experts(block_expert, num_active, xg, W1, b1, W2, b2, gates2d):
    grid_spec = pltpu.PrefetchScalarGridSpec(
        num_scalar_prefetch=2,
        grid=(NB, HS),
        in_specs=[
            pl.BlockSpec((M, C), lambda b, hb, be, na: (b, 0)),
            pl.BlockSpec((1, C, HB), lambda b, hb, be, na: (be[b], 0, hb)),
            pl.BlockSpec((1, 1, HB), lambda b, hb, be, na: (be[b], 0, hb)),
            pl.BlockSpec((1, HB, C), lambda b, hb, be, na: (be[b], hb, 0)),
            pl.BlockSpec((1, 1, C), lambda b, hb, be, na: (be[b], 0, 0)),
            pl.BlockSpec((M, 1), lambda b, hb, be, na: (b, 0)),
        ],
        out_specs=pl.BlockSpec((M, C), lambda b, hb, be, na: (b, 0)),
    )
    return pl.pallas_call(
        _expert_body,
        grid_spec=grid_spec,
        out_shape=jax.ShapeDtypeStruct((P, C), jnp.float32),
        compiler_params=pltpu.CompilerParams(
            dimension_semantics=("arbitrary", "arbitrary"),
        ),
    )(block_expert, num_active, xg, W1, b1, W2, b2, gates2d)


def kernel(x, Wr, br, W1, b1, W2, b2):
    Bv, Tv, Cv = x.shape
    flat_x = x.reshape(N, C)

    eidx, gate = _router(flat_x, Wr, br)

    oh = (eidx[:, None] == jnp.arange(E, dtype=jnp.int32)[None, :]
          ).astype(jnp.int32)
    rank = jnp.take_along_axis(jnp.cumsum(oh, axis=0) - oh,
                               eidx[:, None], axis=1)[:, 0]
    counts = jnp.sum(oh, axis=0)
    nb_e = (counts + (M - 1)) // M
    blk_cum = jnp.cumsum(nb_e)
    blk_start = blk_cum - nb_e
    num_active = blk_cum[E - 1:E]
    slot = blk_start[eidx] * M + rank

    src = jnp.zeros((P,), jnp.int32).at[slot].set(
        jnp.arange(N, dtype=jnp.int32))
    gates_p = jnp.zeros((P,), jnp.float32).at[slot].set(gate)
    block_expert = jnp.minimum(
        jnp.searchsorted(blk_cum, jnp.arange(NB, dtype=jnp.int32),
                         side="right").astype(jnp.int32), E - 1)

    xg = jnp.take(flat_x, src, axis=0)

    y = _experts(block_expert, num_active, xg, W1, b1[:, None, :],
                 W2, b2[:, None, :], gates_p[:, None])

    out = jnp.take(y, slot, axis=0)
    return out.reshape(Bv, Tv, Cv)


# speedup vs baseline: 1.0790x; 1.0790x over previous
import functools

import jax
import jax.numpy as jnp
from jax.experimental import pallas as pl
from jax.experimental.pallas import tpu as pltpu

N = 4096
C = 768
E = 64
H = 3072
M = 128
NB = N // M + E
P = NB * M
RB = 512


def _router_body(x_ref, wr_ref, br_ref, idx_ref, gate_ref):
    logits = jnp.dot(x_ref[...], wr_ref[...],
                     preferred_element_type=jnp.float32) + br_ref[...]
    m = jnp.max(logits, axis=-1, keepdims=True)
    s = jnp.sum(jnp.exp(logits - m), axis=-1, keepdims=True)
    idx_ref[...] = jnp.argmax(logits, axis=-1).astype(jnp.int32)
    gate_ref[...] = (1.0 / s)[:, 0]


def _router(flat_x, Wr, br):
    return pl.pallas_call(
        _router_body,
        grid=(N // RB,),
        in_specs=[
            pl.BlockSpec((RB, C), lambda i: (i, 0)),
            pl.BlockSpec((C, E), lambda i: (0, 0)),
            pl.BlockSpec((E,), lambda i: (0,)),
        ],
        out_specs=[
            pl.BlockSpec((RB,), lambda i: (i,)),
            pl.BlockSpec((RB,), lambda i: (i,)),
        ],
        out_shape=[
            jax.ShapeDtypeStruct((N,), jnp.int32),
            jax.ShapeDtypeStruct((N,), jnp.float32),
        ],
    )(flat_x, Wr, br)


HS = 2
HB = H // HS


def _expert_body(be_ref, na_ref, x_ref, w1_ref, b1_ref, w2_ref, b2_ref,
                 g_ref, y_ref):
    b = pl.program_id(0)
    hb = pl.program_id(1)

    @pl.when(b < na_ref[0])
    def _():
        h = jnp.dot(x_ref[...], w1_ref[0],
                    preferred_element_type=jnp.float32) + b1_ref[0]
        h = 0.5 * h * (1.0 + jax.lax.erf(h * 0.7071067811865476))
        yp = jnp.dot(h, w2_ref[0], preferred_element_type=jnp.float32)

        @pl.when(hb == 0)
        def _():
            y_ref[...] = yp + b2_ref[0]

        @pl.when(hb != 0)
        def _():
            y_ref[...] += yp

        @pl.when(hb == HS - 1)
        def _():
            y_ref[...] *= g_ref[...]


def _experts(block_expert, num_active, xg, W1, b1, W2, b2, gates2d):
    grid_spec = pltpu.PrefetchScalarGridSpec(
        num_scalar_prefetch=2,
        grid=(NB, HS),
        in_specs=[
            pl.BlockSpec((M, C), lambda b, hb, be, na: (b, 0)),
            pl.BlockSpec((1, C, HB), lambda b, hb, be, na: (be[b], 0, hb)),
            pl.BlockSpec((1, 1, HB), lambda b, hb, be, na: (be[b], 0, hb)),
            pl.BlockSpec((1, HB, C), lambda b, hb, be, na: (be[b], hb, 0)),
            pl.BlockSpec((1, 1, C), lambda b, hb, be, na: (be[b], 0, 0)),
            pl.BlockSpec((M, 1), lambda b, hb, be, na: (b, 0)),
        ],
        out_specs=pl.BlockSpec((M, C), lambda b, hb, be, na: (b, 0)),
    )
    return pl.pallas_call(
        _expert_body,
        grid_spec=grid_spec,
        out_shape=jax.ShapeDtypeStruct((P, C), jnp.float32),
        compiler_params=pltpu.CompilerParams(
            dimension_semantics=("arbitrary", "arbitrary"),
        ),
    )(block_expert, num_active, xg, W1, b1, W2, b2, gates2d)


def kernel(x, Wr, br, W1, b1, W2, b2):
    Bv, Tv, Cv = x.shape
    flat_x = x.reshape(N, C)

    eidx, gate = _router(flat_x, Wr, br)

    oh = (eidx[:, None] == jnp.arange(E, dtype=jnp.int32)[None, :]
          ).astype(jnp.int32)
    rank = jnp.take_along_axis(jnp.cumsum(oh, axis=0) - oh,
                               eidx[:, None], axis=1)[:, 0]
    counts = jnp.sum(oh, axis=0)
    nb_e = (counts + (M - 1)) // M
    blk_cum = jnp.cumsum(nb_e)
    blk_start = blk_cum - nb_e
    num_active = blk_cum[E - 1:E]
    slot = blk_start[eidx] * M + rank

    src = jnp.zeros((P,), jnp.int32).at[slot].set(
        jnp.arange(N, dtype=jnp.int32))
    gates_p = jnp.zeros((P,), jnp.float32).at[slot].set(gate)
    block_expert = jnp.minimum(
        jnp.searchsorted(blk_cum, jnp.arange(NB, dtype=jnp.int32),
                         side="right").astype(jnp.int32), E - 1)

    xg = jnp.take(flat_x, src, axis=0)

    y = _experts(block_expert, num_active, xg, W1, b1[:, None, :],
                 W2, b2[:, None, :], gates_p[:, None])

    out = jnp.take(y, slot, axis=0)
    return out.reshape(Bv, Tv, Cv)


# speedup vs baseline: 1.7228x; 1.5966x over previous
import functools

import jax
import jax.numpy as jnp
from jax import lax
from jax.experimental import pallas as pl
from jax.experimental.pallas import tpu as pltpu
from jax.experimental.pallas import tpu_sc as plsc

N = 4096
C = 768
E = 64
H = 3072
M = 128
NB = N // M + E
P = NB * M
RB = 512

NC, NS = 2, 16
NW = NC * NS
TPW = N // NW


def _router_body(x_ref, wr_ref, br_ref, idx_ref, gate_ref, rank_ref,
                 cnt_ref, carry):
    i = pl.program_id(0)

    @pl.when(i == 0)
    def _():
        carry[...] = jnp.zeros_like(carry)

    logits = jnp.dot(x_ref[...], wr_ref[...],
                     preferred_element_type=jnp.float32) + br_ref[...]
    m = jnp.max(logits, axis=-1, keepdims=True)
    s = jnp.sum(jnp.exp(logits - m), axis=-1, keepdims=True)
    idx = jnp.argmax(logits, axis=-1).astype(jnp.int32)
    idx_ref[...] = idx
    gate_ref[...] = (1.0 / s)[:, 0]

    oh = (idx[:, None] == lax.broadcasted_iota(jnp.int32, (RB, E), 1)
          ).astype(jnp.float32)
    row = lax.broadcasted_iota(jnp.int32, (RB, RB), 0)
    col = lax.broadcasted_iota(jnp.int32, (RB, RB), 1)
    lt = (col < row).astype(jnp.float32)
    prior = jnp.dot(lt, oh, preferred_element_type=jnp.float32) + carry[...]
    rank_ref[...] = jnp.sum(oh * prior, axis=-1).astype(jnp.int32)
    new_carry = carry[...] + jnp.sum(oh, axis=0, keepdims=True)
    carry[...] = new_carry

    @pl.when(i == pl.num_programs(0) - 1)
    def _():
        cnt_ref[...] = new_carry[0].astype(jnp.int32)


def _router(flat_x, Wr, br):
    return pl.pallas_call(
        _router_body,
        grid=(N // RB,),
        in_specs=[
            pl.BlockSpec((RB, C), lambda i: (i, 0)),
            pl.BlockSpec((C, E), lambda i: (0, 0)),
            pl.BlockSpec((E,), lambda i: (0,)),
        ],
        out_specs=[
            pl.BlockSpec((RB,), lambda i: (i,)),
            pl.BlockSpec((RB,), lambda i: (i,)),
            pl.BlockSpec((RB,), lambda i: (i,)),
            pl.BlockSpec((E,), lambda i: (0,)),
        ],
        out_shape=[
            jax.ShapeDtypeStruct((N,), jnp.int32),
            jax.ShapeDtypeStruct((N,), jnp.float32),
            jax.ShapeDtypeStruct((N,), jnp.int32),
            jax.ShapeDtypeStruct((E,), jnp.int32),
        ],
        scratch_shapes=[pltpu.VMEM((1, E), jnp.float32)],
        compiler_params=pltpu.CompilerParams(
            dimension_semantics=("arbitrary",),
        ),
    )(flat_x, Wr, br)


@functools.cache
def _sc_mesh():
    return plsc.VectorSubcoreMesh(
        core_axis_name="c", subcore_axis_name="s",
        num_cores=NC, num_subcores=NS)


@functools.cache
def _sc_dispatch():
    @functools.partial(
        pl.kernel,
        out_type=jax.ShapeDtypeStruct((P, C), jnp.float32),
        mesh=_sc_mesh(),
        scratch_types=[
            pltpu.VMEM((TPW,), jnp.int32),
            pltpu.VMEM((TPW, C), jnp.float32),
            pltpu.SemaphoreType.DMA,
        ],
    )
    def dispatch(x_hbm, slot_hbm, padx_hbm, slot_v, rows_v, sem):
        wid = lax.axis_index("s") * NC + lax.axis_index("c")
        base = wid * TPW
        pltpu.sync_copy(slot_hbm.at[pl.ds(base, TPW)], slot_v)
        pltpu.sync_copy(x_hbm.at[pl.ds(base, TPW)], rows_v)
        pltpu.async_copy(rows_v, padx_hbm.at[slot_v], sem).wait()

    return dispatch


@functools.cache
def _sc_gather_back():
    @functools.partial(
        pl.kernel,
        out_type=jax.ShapeDtypeStruct((N, C), jnp.float32),
        mesh=_sc_mesh(),
        scratch_types=[
            pltpu.VMEM((TPW,), jnp.int32),
            pltpu.VMEM((TPW, C), jnp.float32),
            pltpu.SemaphoreType.DMA,
        ],
    )
    def gather_back(y_hbm, slot_hbm, out_hbm, slot_v, rows_v, sem):
        wid = lax.axis_index("s") * NC + lax.axis_index("c")
        base = wid * TPW
        pltpu.sync_copy(slot_hbm.at[pl.ds(base, TPW)], slot_v)
        pltpu.async_copy(y_hbm.at[slot_v], rows_v, sem).wait()
        pltpu.sync_copy(rows_v, out_hbm.at[pl.ds(base, TPW)])

    return gather_back


def _expert_body(be_ref, na_ref, x_ref, w1_ref, b1_ref, w2_ref, b2_ref,
                 y_ref):
    b = pl.program_id(0)

    @pl.when(b < na_ref[0])
    def _():
        h = jnp.dot(x_ref[...], w1_ref[0],
                    preferred_element_type=jnp.float32) + b1_ref[0]
        h = 0.5 * h * (1.0 + lax.erf(h * 0.7071067811865476))
        y_ref[...] = jnp.dot(h, w2_ref[0],
                             preferred_element_type=jnp.float32) + b2_ref[0]


def _experts(block_expert, num_active, xg, W1, b1, W2, b2):
    grid_spec = pltpu.PrefetchScalarGridSpec(
        num_scalar_prefetch=2,
        grid=(NB,),
        in_specs=[
            pl.BlockSpec((M, C), lambda b, be, na: (b, 0)),
            pl.BlockSpec((1, C, H), lambda b, be, na: (be[b], 0, 0)),
            pl.BlockSpec((1, 1, H), lambda b, be, na: (be[b], 0, 0)),
            pl.BlockSpec((1, H, C), lambda b, be, na: (be[b], 0, 0)),
            pl.BlockSpec((1, 1, C), lambda b, be, na: (be[b], 0, 0)),
        ],
        out_specs=pl.BlockSpec((M, C), lambda b, be, na: (b, 0)),
    )
    return pl.pallas_call(
        _expert_body,
        grid_spec=grid_spec,
        out_shape=jax.ShapeDtypeStruct((P, C), jnp.float32),
        compiler_params=pltpu.CompilerParams(
            dimension_semantics=("arbitrary",),
        ),
    )(block_expert, num_active, xg, W1, b1, W2, b2)


def kernel(x, Wr, br, W1, b1, W2, b2):
    Bv, Tv, Cv = x.shape
    flat_x = x.reshape(N, C)

    eidx, gate, rank, counts = _router(flat_x, Wr, br)

    nb_e = (counts + (M - 1)) // M
    blk_cum = jnp.cumsum(nb_e)
    num_active = blk_cum[E - 1:E].astype(jnp.int32)
    pstart = ((blk_cum - nb_e) * M).astype(jnp.int32)
    block_expert = jnp.minimum(
        jnp.searchsorted(blk_cum, jnp.arange(NB, dtype=jnp.int32),
                         side="right").astype(jnp.int32), E - 1)

    slot = pstart[eidx] + rank
    xg = _sc_dispatch()(flat_x, slot)

    y = _experts(block_expert, num_active, xg, W1, b1[:, None, :],
                 W2, b2[:, None, :])

    out = _sc_gather_back()(y, slot)
    out = out * gate[:, None]
    return out.reshape(Bv, Tv, Cv)


# speedup vs baseline: 1.8022x; 1.0461x over previous
import functools

import jax
import jax.numpy as jnp
from jax import lax
from jax.experimental import pallas as pl
from jax.experimental.pallas import tpu as pltpu
from jax.experimental.pallas import tpu_sc as plsc

N = 4096
C = 768
E = 64
H = 3072
M = 128
NB = N // M + E
P = NB * M
RB = 512

NC, NS = 2, 16
NW = NC * NS
TPW = N // NW


def _router_body(x_ref, wr_ref, br_ref, idx_ref, gate_ref, rank_ref,
                 cnt_ref, carry):
    i = pl.program_id(0)

    @pl.when(i == 0)
    def _():
        carry[...] = jnp.zeros_like(carry)

    logits = jnp.dot(x_ref[...], wr_ref[...],
                     preferred_element_type=jnp.float32) + br_ref[...]
    m = jnp.max(logits, axis=-1, keepdims=True)
    s = jnp.sum(jnp.exp(logits - m), axis=-1, keepdims=True)
    idx = jnp.argmax(logits, axis=-1).astype(jnp.int32)
    idx_ref[...] = idx
    gate_ref[...] = (1.0 / s)[:, 0]

    oh = (idx[:, None] == lax.broadcasted_iota(jnp.int32, (RB, E), 1)
          ).astype(jnp.float32)
    row = lax.broadcasted_iota(jnp.int32, (RB, RB), 0)
    col = lax.broadcasted_iota(jnp.int32, (RB, RB), 1)
    lt = (col < row).astype(jnp.float32)
    prior = jnp.dot(lt, oh, preferred_element_type=jnp.float32) + carry[...]
    rank_ref[...] = jnp.sum(oh * prior, axis=-1).astype(jnp.int32)
    new_carry = carry[...] + jnp.sum(oh, axis=0, keepdims=True)
    carry[...] = new_carry

    @pl.when(i == pl.num_programs(0) - 1)
    def _():
        cnt_ref[...] = new_carry[0].astype(jnp.int32)


def _router(flat_x, Wr, br):
    return pl.pallas_call(
        _router_body,
        grid=(N // RB,),
        in_specs=[
            pl.BlockSpec((RB, C), lambda i: (i, 0)),
            pl.BlockSpec((C, E), lambda i: (0, 0)),
            pl.BlockSpec((E,), lambda i: (0,)),
        ],
        out_specs=[
            pl.BlockSpec((RB,), lambda i: (i,)),
            pl.BlockSpec((RB,), lambda i: (i,)),
            pl.BlockSpec((RB,), lambda i: (i,)),
            pl.BlockSpec((E,), lambda i: (0,)),
        ],
        out_shape=[
            jax.ShapeDtypeStruct((N,), jnp.int32),
            jax.ShapeDtypeStruct((N,), jnp.float32),
            jax.ShapeDtypeStruct((N,), jnp.int32),
            jax.ShapeDtypeStruct((E,), jnp.int32),
        ],
        scratch_shapes=[pltpu.VMEM((1, E), jnp.float32)],
        compiler_params=pltpu.CompilerParams(
            dimension_semantics=("arbitrary",),
        ),
    )(flat_x, Wr, br)


@functools.cache
def _sc_mesh():
    return plsc.VectorSubcoreMesh(
        core_axis_name="c", subcore_axis_name="s",
        num_cores=NC, num_subcores=NS)


@functools.cache
def _sc_dispatch():
    @functools.partial(
        pl.kernel,
        out_type=jax.ShapeDtypeStruct((P, C), jnp.float32),
        mesh=_sc_mesh(),
        scratch_types=[
            pltpu.VMEM((TPW,), jnp.int32),
            pltpu.VMEM((TPW, C), jnp.float32),
            pltpu.SemaphoreType.DMA,
        ],
    )
    def dispatch(x_hbm, slot_hbm, padx_hbm, slot_v, rows_v, sem):
        wid = lax.axis_index("s") * NC + lax.axis_index("c")
        base = wid * TPW
        pltpu.sync_copy(slot_hbm.at[pl.ds(base, TPW)], slot_v)
        pltpu.sync_copy(x_hbm.at[pl.ds(base, TPW)], rows_v)
        pltpu.async_copy(rows_v, padx_hbm.at[slot_v], sem).wait()

    return dispatch


@functools.cache
def _sc_gather_back():
    @functools.partial(
        pl.kernel,
        out_type=jax.ShapeDtypeStruct((N, C), jnp.float32),
        mesh=_sc_mesh(),
        scratch_types=[
            pltpu.VMEM((TPW,), jnp.int32),
            pltpu.VMEM((TPW, C), jnp.float32),
            pltpu.SemaphoreType.DMA,
        ],
    )
    def gather_back(y_hbm, slot_hbm, out_hbm, slot_v, rows_v, sem):
        wid = lax.axis_index("s") * NC + lax.axis_index("c")
        base = wid * TPW
        pltpu.sync_copy(slot_hbm.at[pl.ds(base, TPW)], slot_v)
        pltpu.async_copy(y_hbm.at[slot_v], rows_v, sem).wait()
        pltpu.sync_copy(rows_v, out_hbm.at[pl.ds(base, TPW)])

    return gather_back


def _expert_body(be_ref, x_ref, w1_ref, b1_ref, w2_ref, b2_ref, y_ref):
    h = jnp.dot(x_ref[...], w1_ref[0],
                preferred_element_type=jnp.float32) + b1_ref[0]
    h = 0.5 * h * (1.0 + lax.erf(h * 0.7071067811865476))
    y_ref[...] = jnp.dot(h, w2_ref[0],
                         preferred_element_type=jnp.float32) + b2_ref[0]


def _experts(block_expert, num_active, xg, W1, b1, W2, b2):
    grid_spec = pltpu.PrefetchScalarGridSpec(
        num_scalar_prefetch=1,
        grid=(num_active,),
        in_specs=[
            pl.BlockSpec((M, C), lambda b, be: (b, 0)),
            pl.BlockSpec((1, C, H), lambda b, be: (be[b], 0, 0)),
            pl.BlockSpec((1, 1, H), lambda b, be: (be[b], 0, 0)),
            pl.BlockSpec((1, H, C), lambda b, be: (be[b], 0, 0)),
            pl.BlockSpec((1, 1, C), lambda b, be: (be[b], 0, 0)),
        ],
        out_specs=pl.BlockSpec((M, C), lambda b, be: (b, 0)),
    )
    return pl.pallas_call(
        _expert_body,
        grid_spec=grid_spec,
        out_shape=jax.ShapeDtypeStruct((P, C), jnp.float32),
        compiler_params=pltpu.CompilerParams(
            dimension_semantics=("arbitrary",),
        ),
    )(block_expert, xg, W1, b1, W2, b2)


def kernel(x, Wr, br, W1, b1, W2, b2):
    Bv, Tv, Cv = x.shape
    flat_x = x.reshape(N, C)

    eidx, gate, rank, counts = _router(flat_x, Wr, br)

    nb_e = (counts + (M - 1)) // M
    blk_cum = jnp.cumsum(nb_e)
    num_active = blk_cum[E - 1].astype(jnp.int32)
    pstart = ((blk_cum - nb_e) * M).astype(jnp.int32)
    block_expert = jnp.minimum(
        jnp.searchsorted(blk_cum, jnp.arange(NB, dtype=jnp.int32),
                         side="right").astype(jnp.int32), E - 1)

    slot = pstart[eidx] + rank
    xg = _sc_dispatch()(flat_x, slot)

    y = _experts(block_expert, num_active, xg, W1, b1[:, None, :],
                 W2, b2[:, None, :])

    out = _sc_gather_back()(y, slot)
    out = out * gate[:, None]
    return out.reshape(Bv, Tv, Cv)
